# Initial kernel scaffold; baseline (speedup 1.0000x reference)
import functools

import jax
import jax.numpy as jnp
from jax import lax
from jax.experimental import pallas as pl
from jax.experimental.pallas import tpu as pltpu
from jax.experimental.pallas import tpu_sc as plsc

_EMB = 512
_NW = 32
_CHUNK = 112


@functools.partial(jax.jit, static_argnames=("n_chunks",))
def _sc_embedding_gather(idx, table, n_chunks):
    n_rows = _NW * n_chunks * _CHUNK
    bpw = n_chunks * _CHUNK
    mesh = plsc.VectorSubcoreMesh(core_axis_name="c", subcore_axis_name="s")

    @functools.partial(
        pl.kernel,
        mesh=mesh,
        out_type=jax.ShapeDtypeStruct((n_rows, _EMB), jnp.float32),
        scratch_types=[
            pltpu.VMEM((n_chunks, _CHUNK), jnp.int32),
            pltpu.VMEM((2, _CHUNK, _EMB), jnp.float32),
            pltpu.SemaphoreType.DMA,
            pltpu.SemaphoreType.DMA,
        ],
    )
    def k(idx_hbm, table_hbm, out_hbm, idx_v, rows_v, gsem, ssem):
        wid = lax.axis_index("s") * 2 + lax.axis_index("c")
        base = wid * bpw
        pltpu.sync_copy(idx_hbm.at[wid], idx_v)

        def gather(j, b):
            return pltpu.async_copy(
                table_hbm.at[idx_v.at[j]], rows_v.at[b], gsem)

        def scatter(j, b):
            return pltpu.async_copy(
                rows_v.at[b], out_hbm.at[pl.ds(base + j * _CHUNK, _CHUNK)],
                ssem)

        g = {0: gather(0, 0)}
        s = {}
        for j in range(n_chunks):
            b = j & 1
            g[j].wait()
            if j + 1 < n_chunks:
                if j >= 1:
                    s[j - 1].wait()
                g[j + 1] = gather(j + 1, (j + 1) & 1)
            s[j] = scatter(j, b)
        s[n_chunks - 1].wait()

    return k(idx, table)


def kernel(text, W):
    batch, ctx = text.shape
    n = batch * ctx
    n_chunks = n // (_NW * _CHUNK)
    idx = text.astype(jnp.int32).reshape(_NW, n_chunks, _CHUNK)
    out = _sc_embedding_gather(idx, W, n_chunks)
    return out.reshape(batch, ctx, _EMB).astype(jnp.float32)


# speedup vs baseline: 1.2604x; 1.2604x over previous
import functools

import jax
import jax.numpy as jnp
from jax import lax
from jax.experimental import pallas as pl
from jax.experimental.pallas import tpu as pltpu
from jax.experimental.pallas import tpu_sc as plsc

_EMB = 512
_NW = 32
_CHUNK = 112


@functools.partial(jax.jit, static_argnames=("n_chunks",))
def _sc_embedding_gather(idx, table, n_chunks):
    n_rows = _NW * n_chunks * _CHUNK
    bpw = n_chunks * _CHUNK
    mesh = plsc.VectorSubcoreMesh(core_axis_name="c", subcore_axis_name="s")

    @functools.partial(
        pl.kernel,
        mesh=mesh,
        out_type=jax.ShapeDtypeStruct((n_rows, _EMB), jnp.float32),
        scratch_types=[
            pltpu.VMEM((n_chunks, _CHUNK), jnp.int32),
            pltpu.VMEM((2, _CHUNK, _EMB), jnp.float32),
            pltpu.SemaphoreType.DMA,
            pltpu.SemaphoreType.DMA,
        ],
    )
    def k(idx_hbm, table_hbm, out_hbm, idx_v, rows_v, gsem, ssem):
        wid = lax.axis_index("s") * 2 + lax.axis_index("c")
        base = wid * bpw
        pltpu.sync_copy(idx_hbm.at[wid], idx_v)

        def gather(j, b):
            return pltpu.async_copy(
                table_hbm.at[idx_v.at[j]], rows_v.at[b], gsem)

        def scatter(j, b):
            return pltpu.async_copy(
                rows_v.at[b], out_hbm.at[pl.ds(base + j * _CHUNK, _CHUNK)],
                ssem)

        g = {0: gather(0, 0)}
        s = {}
        for j in range(n_chunks):
            b = j & 1
            g[j].wait()
            if j + 1 < n_chunks:
                if j >= 1:
                    s[j - 1].wait()
                g[j + 1] = gather(j + 1, (j + 1) & 1)
            s[j] = scatter(j, b)
        if n_chunks >= 2:
            s[n_chunks - 2].wait()
        s[n_chunks - 1].wait()

    return k(idx, table)


def kernel(text, W):
    batch, ctx = text.shape
    n = batch * ctx
    n_chunks = n // (_NW * _CHUNK)
    idx = text.astype(jnp.int32).reshape(_NW, n_chunks, _CHUNK)
    out = _sc_embedding_gather(idx, W, n_chunks)
    return out.reshape(batch, ctx, _EMB).astype(jnp.float32)


# speedup vs baseline: 1.2659x; 1.0044x over previous
import functools

import jax
import jax.numpy as jnp
from jax import lax
from jax.experimental import pallas as pl
from jax.experimental.pallas import tpu as pltpu
from jax.experimental.pallas import tpu_sc as plsc

_EMB = 512
_NW = 32
_CHUNK = 56
_NBUF = 4


@functools.partial(jax.jit, static_argnames=("n_chunks",))
def _sc_embedding_gather(idx, table, n_chunks):
    n_rows = _NW * n_chunks * _CHUNK
    bpw = n_chunks * _CHUNK
    mesh = plsc.VectorSubcoreMesh(core_axis_name="c", subcore_axis_name="s")

    @functools.partial(
        pl.kernel,
        mesh=mesh,
        out_type=jax.ShapeDtypeStruct((n_rows, _EMB), jnp.float32),
        scratch_types=[
            pltpu.VMEM((n_chunks, _CHUNK), jnp.int32),
            pltpu.VMEM((_NBUF, _CHUNK, _EMB), jnp.float32),
        ] + [pltpu.SemaphoreType.DMA] * (2 * _NBUF),
    )
    def k(idx_hbm, table_hbm, out_hbm, idx_v, rows_v, *sems):
        gsem, ssem = sems[:_NBUF], sems[_NBUF:]
        wid = lax.axis_index("s") * 2 + lax.axis_index("c")
        base = wid * bpw
        pltpu.sync_copy(idx_hbm.at[wid], idx_v)

        def gather(j):
            b = j % _NBUF
            return pltpu.async_copy(
                table_hbm.at[idx_v.at[j]], rows_v.at[b], gsem[b])

        def scatter(j):
            b = j % _NBUF
            return pltpu.async_copy(
                rows_v.at[b], out_hbm.at[pl.ds(base + j * _CHUNK, _CHUNK)],
                ssem[b])

        g = {j: gather(j) for j in range(min(_NBUF - 1, n_chunks))}
        s = {}
        for j in range(n_chunks):
            g[j].wait()
            s[j] = scatter(j)
            nxt = j + _NBUF - 1
            if nxt < n_chunks:
                if j >= 1:
                    s[j - 1].wait()
                g[nxt] = gather(nxt)
        for j in range(max(0, n_chunks - _NBUF), n_chunks):
            s[j].wait()

    return k(idx, table)


def kernel(text, W):
    batch, ctx = text.shape
    n = batch * ctx
    n_chunks = n // (_NW * _CHUNK)
    idx = text.astype(jnp.int32).reshape(_NW, n_chunks, _CHUNK)
    out = _sc_embedding_gather(idx, W, n_chunks)
    return out.reshape(batch, ctx, _EMB).astype(jnp.float32)


# speedup vs baseline: 3.6687x; 2.8980x over previous
import functools

import jax
import jax.numpy as jnp
from jax import lax
from jax.experimental import pallas as pl
from jax.experimental.pallas import tpu as pltpu
from jax.experimental.pallas import tpu_sc as plsc

_EMB = 512
_NW = 32
_CHUNK = 56
_NBUF = 4


@functools.partial(jax.jit, static_argnames=("n_chunks",))
def _sc_embedding_gather(idx, table, n_chunks):
    n_rows = _NW * n_chunks * _CHUNK
    bpw = n_chunks * _CHUNK
    mesh = plsc.VectorSubcoreMesh(core_axis_name="c", subcore_axis_name="s")

    @functools.partial(
        pl.kernel,
        mesh=mesh,
        out_type=jax.ShapeDtypeStruct((n_rows, _EMB), jnp.float32),
        scratch_types=[
            pltpu.VMEM((n_chunks, _CHUNK), jnp.int32),
            pltpu.VMEM((_NBUF, _CHUNK, _EMB), jnp.float32),
        ] + [pltpu.SemaphoreType.DMA] * (2 * _NBUF),
    )
    def k(idx_hbm, table_hbm, out_hbm, idx_v, rows_v, *sems):
        gsem, ssem = sems[:_NBUF], sems[_NBUF:]
        wid = lax.axis_index("s") * 2 + lax.axis_index("c")
        base = wid * bpw
        pltpu.sync_copy(idx_hbm.at[wid], idx_v)

        def gather(j):
            b = j % _NBUF
            return pltpu.async_copy(
                table_hbm.at[idx_v.at[j]], rows_v.at[b], gsem[b])

        def scatter(j):
            b = j % _NBUF
            return pltpu.async_copy(
                rows_v.at[b], out_hbm.at[pl.ds(base + j * _CHUNK, _CHUNK)],
                ssem[b])

        g = {j: gather(j) for j in range(min(_NBUF - 1, n_chunks))}
        s = {}
        for j in range(n_chunks):
            g[j].wait()
            s[j] = scatter(j)
            nxt = j + _NBUF - 1
            if nxt < n_chunks:
                if j >= 1:
                    s[j - 1].wait()
                g[nxt] = gather(nxt)
        for j in range(max(0, n_chunks - _NBUF), n_chunks):
            s[j].wait()

    return k(idx, table)


def kernel(text, W):
    batch, ctx = text.shape
    n = batch * ctx
    n_chunks = n // (_NW * _CHUNK)
    idx = text.T.astype(jnp.int32).reshape(_NW, n_chunks, _CHUNK)
    out = _sc_embedding_gather(idx, W, n_chunks)
    return out.reshape(ctx, batch, _EMB).swapaxes(0, 1)
